# baseline (speedup 1.0000x reference)
import jax
import jax.numpy as jnp
from jax.experimental import pallas as pl
from jax.experimental.pallas import tpu as pltpu

N_TOKENS = 8192
D_MODEL = 2048
NUM_EXPERTS = 16
CAPACITY = float(N_TOKENS)
EPS = 1e-6
BLK = 1024


def _router_body(x_ref, wt_ref, out_ref, denom_ref):
    i = pl.program_id(0)
    scores = jax.lax.dot_general(
        x_ref[...], wt_ref[...], (((1,), (1,)), ((), ())),
        preferred_element_type=jnp.float32)
    rowmax = jnp.max(scores, axis=-1, keepdims=True)
    col_rev = jax.lax.broadcasted_iota(jnp.int32, scores.shape, 1)
    col_rev = (NUM_EXPERTS - 1) - col_rev
    enc = jnp.where(scores == rowmax, col_rev, -1)
    best = jnp.max(enc, axis=-1, keepdims=True)
    masked = jnp.where(enc == best, scores, 0.0)
    psum = jnp.sum(masked, axis=0, keepdims=True)

    @pl.when(i == 0)
    def _init():
        denom_ref[...] = psum

    @pl.when(i > 0)
    def _acc():
        denom_ref[...] += psum

    out_ref[pl.ds(i * BLK, BLK), :] = masked

    @pl.when(i == pl.num_programs(0) - 1)
    def _normalize():
        out_ref[...] = out_ref[...] * (CAPACITY / (denom_ref[...] + EPS))


def kernel(x, w_gate):
    grid = (N_TOKENS // BLK,)
    return pl.pallas_call(
        _router_body,
        grid=grid,
        in_specs=[
            pl.BlockSpec((BLK, D_MODEL), lambda i: (i, 0)),
            pl.BlockSpec((NUM_EXPERTS, D_MODEL), lambda i: (0, 0)),
        ],
        out_specs=pl.BlockSpec((N_TOKENS, NUM_EXPERTS), lambda i: (0, 0)),
        out_shape=jax.ShapeDtypeStruct((N_TOKENS, NUM_EXPERTS), jnp.float32),
        scratch_shapes=[pltpu.VMEM((1, NUM_EXPERTS), jnp.float32)],
    )(x, w_gate)
